# baseline (speedup 1.0000x reference)
import jax
import jax.numpy as jnp
from jax.experimental import pallas as pl
from jax.experimental.pallas import tpu as pltpu

_B, _L, _NA, _EMB, _HID = 64, 128, 16, 256, 64
_N = _B * _L
_BB = 32
_NB = _BB * _L
_NS = 1 + _NA


def _body(fwd_a, fwd_b, bwd_a, bwd_b, rew_ref, pif_ref, ifl_ref, w1_ref,
          b1_ref, w2_ref, b2_ref, ones_ref, out_ref, z_scr):
    j = pl.program_id(1)

    def head(x4):
        x = x4.reshape(_NB // 2, _EMB)
        h = jnp.tanh(
            jnp.dot(x, w1_ref[...], preferred_element_type=jnp.float32)
            + b1_ref[...])
        return jnp.dot(h, w2_ref[...], preferred_element_type=jnp.float32)

    @pl.when(j == 0)
    def _():
        yf = jnp.concatenate([head(fwd_a[...]), head(fwd_b[...])], axis=0)
        fout = jnp.dot(jax.nn.softplus(yf + b2_ref[...]), ones_ref[...],
                       preferred_element_type=jnp.float32)
        out_ref[:, 1:2] = fout
        out_ref[:, 2:3] = rew_ref[...]
        out_ref[:, 3:4] = pif_ref[...] * jnp.exp(ifl_ref[...])

    @pl.when(j > 0)
    def _():
        y = jnp.concatenate([head(bwd_a[...]), head(bwd_b[...])], axis=0)
        lane = jax.lax.broadcasted_iota(jnp.int32, (_NB, _NA), 1)
        z_scr[...] = jnp.where(lane == j - 1, y, z_scr[...])

    @pl.when(j == _NS - 1)
    def _():
        fin = jnp.dot(jax.nn.softplus(z_scr[...] + b2_ref[...]),
                      ones_ref[...],
                      preferred_element_type=jnp.float32)
        out_ref[:, 0:1] = fin


def kernel(forward_edges, backward_edges, paths_reward, path_init_flow,
           initial_flow, W1, b1, W2, b2):
    fwdT = jnp.transpose(forward_edges, (0, 2, 1, 3))
    bwdT = jnp.transpose(backward_edges, (0, 2, 1, 3))
    rew = paths_reward.reshape(_N, 1)
    pif = path_init_flow.reshape(_N, 1)
    ifl = initial_flow.reshape(1, 1)
    b1r = b1.reshape(1, _HID)
    b2r = b2.reshape(1, _NA)
    ones = jnp.ones((_NA, 1), jnp.float32)

    out = pl.pallas_call(
        _body,
        grid=(_B // _BB, _NS),
        in_specs=[
            pl.BlockSpec((_BB // 2, 1, _L, _EMB),
                         lambda i, j: (2 * i, 0, 0, 0)),
            pl.BlockSpec((_BB // 2, 1, _L, _EMB),
                         lambda i, j: (2 * i + 1, 0, 0, 0)),
            pl.BlockSpec((_BB // 2, 1, _L, _EMB),
                         lambda i, j: (2 * i, jnp.maximum(j, 1), 0, 0)),
            pl.BlockSpec((_BB // 2, 1, _L, _EMB),
                         lambda i, j: (2 * i + 1, jnp.maximum(j, 1), 0, 0)),
            pl.BlockSpec((_NB, 1), lambda i, j: (i, 0)),
            pl.BlockSpec((_NB, 1), lambda i, j: (i, 0)),
            pl.BlockSpec((1, 1), lambda i, j: (0, 0)),
            pl.BlockSpec((_EMB, _HID), lambda i, j: (0, 0)),
            pl.BlockSpec((1, _HID), lambda i, j: (0, 0)),
            pl.BlockSpec((_HID, _NA), lambda i, j: (0, 0)),
            pl.BlockSpec((1, _NA), lambda i, j: (0, 0)),
            pl.BlockSpec((_NA, 1), lambda i, j: (0, 0)),
        ],
        out_specs=pl.BlockSpec((_NB, 4), lambda i, j: (i, 0)),
        out_shape=jax.ShapeDtypeStruct((_N, 4), jnp.float32),
        scratch_shapes=[
            pltpu.VMEM((_NB, _NA), jnp.float32),
        ],
        compiler_params=pltpu.CompilerParams(
            dimension_semantics=("parallel", "arbitrary")),
    )(fwdT, fwdT, bwdT, bwdT, rew, pif, ifl, W1, b1r, W2, b2r, ones)

    return out.reshape(_B, _L, 4)


# speedup vs baseline: 1.1582x; 1.1582x over previous
import jax
import jax.numpy as jnp
from jax.experimental import pallas as pl
from jax.experimental.pallas import tpu as pltpu

_B, _L, _NA, _EMB, _HID = 64, 128, 16, 256, 64
_N = _B * _L
_NS = 1 + _NA


def _body(fwd_hbm, bwd_ref, rew_ref, pif_ref, ifl_ref, w1_ref, b1_ref,
          w2_ref, b2_ref, ones_ref, out_ref, fscr, z_scr, sem):
    j = pl.program_id(0)

    def fwd_copy():
        return pltpu.make_async_copy(fwd_hbm.at[:, 0, :, :], fscr, sem)

    @pl.when(j == 0)
    def _():
        fwd_copy().start()

    def head(x4):
        x = x4.reshape(_N, _EMB)
        h = jnp.tanh(
            jnp.dot(x, w1_ref[...], preferred_element_type=jnp.float32)
            + b1_ref[...])
        return jnp.dot(h, w2_ref[...], preferred_element_type=jnp.float32)

    @pl.when(j < _NS - 1)
    def _():
        y = head(bwd_ref[...])
        lane = jax.lax.broadcasted_iota(jnp.int32, (_N, _NA), 1)
        z_scr[...] = jnp.where(lane == j, y, z_scr[...])

    @pl.when(j == _NS - 1)
    def _():
        fin = jnp.dot(jax.nn.softplus(z_scr[...] + b2_ref[...]),
                      ones_ref[...],
                      preferred_element_type=jnp.float32)
        fwd_copy().wait()
        yf = head(fscr[...])
        fout = jnp.dot(jax.nn.softplus(yf + b2_ref[...]), ones_ref[...],
                       preferred_element_type=jnp.float32)
        out_ref[:, 0:1] = fin
        out_ref[:, 1:2] = fout
        out_ref[:, 2:3] = rew_ref[...]
        out_ref[:, 3:4] = pif_ref[...] * jnp.exp(ifl_ref[...])


def kernel(forward_edges, backward_edges, paths_reward, path_init_flow,
           initial_flow, W1, b1, W2, b2):
    fwdT = jnp.transpose(forward_edges, (0, 2, 1, 3))
    bwdT = jnp.transpose(backward_edges, (0, 2, 1, 3))
    rew = paths_reward.reshape(_N, 1)
    pif = path_init_flow.reshape(_N, 1)
    ifl = initial_flow.reshape(1, 1)
    b1r = b1.reshape(1, _HID)
    b2r = b2.reshape(1, _NA)
    ones = jnp.ones((_NA, 1), jnp.float32)

    out = pl.pallas_call(
        _body,
        grid=(_NS,),
        in_specs=[
            pl.BlockSpec(memory_space=pl.ANY),
            pl.BlockSpec((_B, 1, _L, _EMB),
                         lambda j: (0, jnp.minimum(j + 1, _NS - 1), 0, 0)),
            pl.BlockSpec((_N, 1), lambda j: (0, 0)),
            pl.BlockSpec((_N, 1), lambda j: (0, 0)),
            pl.BlockSpec((1, 1), lambda j: (0, 0)),
            pl.BlockSpec((_EMB, _HID), lambda j: (0, 0)),
            pl.BlockSpec((1, _HID), lambda j: (0, 0)),
            pl.BlockSpec((_HID, _NA), lambda j: (0, 0)),
            pl.BlockSpec((1, _NA), lambda j: (0, 0)),
            pl.BlockSpec((_NA, 1), lambda j: (0, 0)),
        ],
        out_specs=pl.BlockSpec((_N, 4), lambda j: (0, 0)),
        out_shape=jax.ShapeDtypeStruct((_N, 4), jnp.float32),
        scratch_shapes=[
            pltpu.VMEM((_B, _L, _EMB), jnp.float32),
            pltpu.VMEM((_N, _NA), jnp.float32),
            pltpu.SemaphoreType.DMA,
        ],
        compiler_params=pltpu.CompilerParams(
            dimension_semantics=("arbitrary",)),
    )(fwdT, bwdT, rew, pif, ifl, W1, b1r, W2, b2r, ones)

    return out.reshape(_B, _L, 4)


# speedup vs baseline: 1.2126x; 1.0469x over previous
import jax
import jax.numpy as jnp
from jax.experimental import pallas as pl
from jax.experimental.pallas import tpu as pltpu

_B, _L, _NA, _EMB, _HID = 64, 128, 16, 256, 64
_N = _B * _L
_NS = 1 + _NA


def _body(fwd_hbm, bwd_ref, rew_ref, pif_ref, ifl_ref, w1_ref, b1_ref,
          w2_ref, w2t_ref, b2_ref, ones_ref, out_ref, fscr, z_scr, sem):
    j = pl.program_id(0)

    def fwd_copy():
        return pltpu.make_async_copy(fwd_hbm.at[:, 0, :, :], fscr, sem)

    @pl.when(j == 0)
    def _():
        fwd_copy().start()

    def head(x4):
        x = x4.reshape(_N, _EMB)
        h = jnp.tanh(
            jnp.dot(x, w1_ref[...], preferred_element_type=jnp.float32)
            + b1_ref[...])
        return jnp.dot(h, w2_ref[...], preferred_element_type=jnp.float32)

    @pl.when(j < _NS - 1)
    def _():
        x = bwd_ref[...].reshape(_N, _EMB)
        h = jnp.tanh(
            jnp.dot(x, w1_ref[...], preferred_element_type=jnp.float32)
            + b1_ref[...])
        w2col = jnp.transpose(w2t_ref[pl.ds(j, 1), :])
        ycol = jnp.dot(h, w2col, preferred_element_type=jnp.float32)
        for s in range(_NA):
            @pl.when(j == s)
            def _():
                z_scr[:, s:s + 1] = ycol

    @pl.when(j == _NS - 1)
    def _():
        fin = jnp.dot(jax.nn.softplus(z_scr[...] + b2_ref[...]),
                      ones_ref[...],
                      preferred_element_type=jnp.float32)
        fwd_copy().wait()
        yf = head(fscr[...])
        fout = jnp.dot(jax.nn.softplus(yf + b2_ref[...]), ones_ref[...],
                       preferred_element_type=jnp.float32)
        out_ref[:, 0:1] = fin
        out_ref[:, 1:2] = fout
        out_ref[:, 2:3] = rew_ref[...]
        out_ref[:, 3:4] = pif_ref[...] * jnp.exp(ifl_ref[...])


def kernel(forward_edges, backward_edges, paths_reward, path_init_flow,
           initial_flow, W1, b1, W2, b2):
    fwdT = jnp.transpose(forward_edges, (0, 2, 1, 3))
    bwdT = jnp.transpose(backward_edges, (0, 2, 1, 3))
    rew = paths_reward.reshape(_N, 1)
    pif = path_init_flow.reshape(_N, 1)
    ifl = initial_flow.reshape(1, 1)
    b1r = b1.reshape(1, _HID)
    b2r = b2.reshape(1, _NA)
    ones = jnp.ones((_NA, 1), jnp.float32)

    out = pl.pallas_call(
        _body,
        grid=(_NS,),
        in_specs=[
            pl.BlockSpec(memory_space=pl.ANY),
            pl.BlockSpec((_B, 1, _L, _EMB),
                         lambda j: (0, jnp.minimum(j + 1, _NS - 1), 0, 0)),
            pl.BlockSpec((_N, 1), lambda j: (0, 0)),
            pl.BlockSpec((_N, 1), lambda j: (0, 0)),
            pl.BlockSpec((1, 1), lambda j: (0, 0)),
            pl.BlockSpec((_EMB, _HID), lambda j: (0, 0)),
            pl.BlockSpec((1, _HID), lambda j: (0, 0)),
            pl.BlockSpec((_HID, _NA), lambda j: (0, 0)),
            pl.BlockSpec((_NA, _HID), lambda j: (0, 0)),
            pl.BlockSpec((1, _NA), lambda j: (0, 0)),
            pl.BlockSpec((_NA, 1), lambda j: (0, 0)),
        ],
        out_specs=pl.BlockSpec((_N, 4), lambda j: (0, 0)),
        out_shape=jax.ShapeDtypeStruct((_N, 4), jnp.float32),
        scratch_shapes=[
            pltpu.VMEM((_B, _L, _EMB), jnp.float32),
            pltpu.VMEM((_N, _NA), jnp.float32),
            pltpu.SemaphoreType.DMA,
        ],
        compiler_params=pltpu.CompilerParams(
            dimension_semantics=("arbitrary",)),
    )(fwdT, bwdT, rew, pif, ifl, W1, b1r, W2, W2.T, b2r, ones)

    return out.reshape(_B, _L, 4)
